# Initial kernel scaffold; baseline (speedup 1.0000x reference)
import jax
import jax.numpy as jnp
from jax.experimental import pallas as pl


def kernel(preds, masks, keypoints, gt_tags, heatmaps):
    raise NotImplementedError("write your pallas kernel here")


# speedup vs baseline: 2.2941x; 2.2941x over previous
import functools

import jax
import jax.numpy as jnp
from jax import lax
from jax.experimental import pallas as pl
from jax.experimental.pallas import tpu as pltpu
from jax.experimental.pallas import tpu_sc as plsc

_LANES = 16


def _make_tag_kernel(n_workers, n_chunks, chans, n_parts, hw):
    mesh = plsc.VectorSubcoreMesh(core_axis_name="c", subcore_axis_name="s")

    @functools.partial(
        pl.kernel,
        mesh=mesh,
        out_type=jax.ShapeDtypeStruct((n_workers, _LANES), jnp.float32),
        scratch_types=[
            pltpu.VMEM((n_chunks, 128), jnp.int32),
            pltpu.VMEM((n_chunks, 128), jnp.int32),
            pltpu.VMEM((n_chunks, 128), jnp.float32),
            pltpu.VMEM((n_chunks, 128), jnp.float32),
            pltpu.VMEM((n_chunks, 128), jnp.float32),
            pltpu.VMEM((_LANES,), jnp.float32),
            pltpu.SemaphoreType.DMA,
        ],
    )
    def tag_kernel(preds_flat, kp_idx, gt, vis, out,
                   idx_v, addr_v, vals_v, gt_v, vis_v, acc_v, sem):
        wid = lax.axis_index("s") * 2 + lax.axis_index("c")
        b = wid // 2
        pltpu.sync_copy(kp_idx.at[b], idx_v)
        pltpu.sync_copy(gt.at[b], gt_v)
        pltpu.sync_copy(vis.at[b], vis_v)
        acc = jnp.zeros((_LANES,), jnp.float32)
        for t in range(2):
            j = wid * 2 + t
            base = (j * chans + n_parts) * hw
            for c in range(n_chunks):
                for i in range(128 // _LANES):
                    sl = pl.ds(i * _LANES, _LANES)
                    addr_v[c, sl] = idx_v[c, sl] + base
            copies = [
                pltpu.async_copy(preds_flat.at[addr_v.at[c]], vals_v.at[c], sem)
                for c in range(n_chunks)
            ]
            for cp in copies:
                cp.wait()
            for c in range(n_chunks):
                for i in range(128 // _LANES):
                    sl = pl.ds(i * _LANES, _LANES)
                    d = vals_v[c, sl] - gt_v[c, sl]
                    acc = acc + d * d * vis_v[c, sl]
        acc_v[...] = acc
        pltpu.sync_copy(acc_v, out.at[wid])

    return tag_kernel


def _det_body(tag_scale, det_scale, preds_ref, heat_ref, mask_ref,
              part_ref, out_ref):
    b = pl.program_id(0)
    s = pl.program_id(1)
    d = preds_ref[0, 0]
    h = heat_ref[0]
    m = mask_ref[0]
    psum = jnp.sum((d - h) ** 2 * m[None, :, :])

    @pl.when(jnp.logical_and(b == 0, s == 0))
    def _():
        out_ref[0, 0] = jnp.sum(part_ref[...]) * tag_scale

    out_ref[0, 0] = out_ref[0, 0] + psum * det_scale


def kernel(preds, masks, keypoints, gt_tags, heatmaps):
    loss_weights = (0.001, 1.0)
    B, nstack, chans, H, W = preds.shape
    n_parts = heatmaps.shape[1]
    tag_dim = gt_tags.shape[1]
    P, K = keypoints.shape[1], keypoints.shape[2]
    pk = P * K
    n_chunks = -(-pk // 128)
    pk_pad = n_chunks * 128
    n_workers = 32
    hw = H * W

    preds_flat = preds.reshape(-1)
    idx = keypoints[..., 0].astype(jnp.int32).reshape(B, pk)
    vis = keypoints[..., 1].astype(jnp.float32).reshape(B, pk)
    gt = gt_tags.astype(jnp.float32).reshape(B, pk)
    pad = ((0, 0), (0, pk_pad - pk))
    idx = jnp.pad(idx, pad).reshape(B, n_chunks, 128)
    vis = jnp.pad(vis, pad).reshape(B, n_chunks, 128)
    gt = jnp.pad(gt, pad).reshape(B, n_chunks, 128)

    tag_kernel = _make_tag_kernel(n_workers, n_chunks, chans, n_parts, hw)
    partials = tag_kernel(preds_flat, idx, gt, vis)

    tag_scale = loss_weights[0] / (B * nstack * tag_dim)
    det_scale = loss_weights[1] / (B * nstack * n_parts * H * W)
    out = pl.pallas_call(
        functools.partial(_det_body, tag_scale, det_scale),
        grid=(B, nstack),
        in_specs=[
            pl.BlockSpec((1, 1, n_parts, H, W), lambda b, s: (b, s, 0, 0, 0)),
            pl.BlockSpec((1, n_parts, H, W), lambda b, s: (b, 0, 0, 0)),
            pl.BlockSpec((1, H, W), lambda b, s: (b, 0, 0)),
            pl.BlockSpec((n_workers, _LANES), lambda b, s: (0, 0)),
        ],
        out_specs=pl.BlockSpec(memory_space=pltpu.SMEM),
        out_shape=jax.ShapeDtypeStruct((1, 1), jnp.float32),
        compiler_params=pltpu.CompilerParams(
            dimension_semantics=("arbitrary", "arbitrary")),
    )(preds, heatmaps, masks, partials)
    return out[0, 0]


# speedup vs baseline: 3.3914x; 1.4783x over previous
import functools

import jax
import jax.numpy as jnp
from jax import lax
from jax.experimental import pallas as pl
from jax.experimental.pallas import tpu as pltpu
from jax.experimental.pallas import tpu_sc as plsc

_LANES = 16


def _make_tag_kernel(n_workers, n_chunks, chans, n_parts, hw):
    mesh = plsc.VectorSubcoreMesh(core_axis_name="c", subcore_axis_name="s")

    @functools.partial(
        pl.kernel,
        mesh=mesh,
        out_type=jax.ShapeDtypeStruct((n_workers, _LANES), jnp.float32),
        scratch_types=[
            pltpu.VMEM((n_chunks, 128), jnp.int32),
            pltpu.VMEM((n_chunks, 128), jnp.int32),
            pltpu.VMEM((n_chunks, 128), jnp.float32),
            pltpu.VMEM((n_chunks, 128), jnp.float32),
            pltpu.VMEM((n_chunks, 128), jnp.float32),
            pltpu.VMEM((_LANES,), jnp.float32),
            pltpu.SemaphoreType.DMA,
        ],
    )
    def tag_kernel(preds_flat, kp_idx, gt, vis, out,
                   idx_v, addr_v, vals_v, gt_v, vis_v, acc_v, sem):
        wid = lax.axis_index("s") * 2 + lax.axis_index("c")
        b = wid // 2
        pltpu.sync_copy(kp_idx.at[b], idx_v)
        pltpu.sync_copy(gt.at[b], gt_v)
        pltpu.sync_copy(vis.at[b], vis_v)
        acc = jnp.zeros((_LANES,), jnp.float32)
        for t in range(2):
            j = wid * 2 + t
            base = (j * chans + n_parts) * hw
            for c in range(n_chunks):
                for i in range(128 // _LANES):
                    sl = pl.ds(i * _LANES, _LANES)
                    addr_v[c, sl] = idx_v[c, sl] + base
            copies = [
                pltpu.async_copy(preds_flat.at[addr_v.at[c]], vals_v.at[c], sem)
                for c in range(n_chunks)
            ]
            for cp in copies:
                cp.wait()
            for c in range(n_chunks):
                for i in range(128 // _LANES):
                    sl = pl.ds(i * _LANES, _LANES)
                    d = vals_v[c, sl] - gt_v[c, sl]
                    acc = acc + d * d * vis_v[c, sl]
        acc_v[...] = acc
        pltpu.sync_copy(acc_v, out.at[wid])

    return tag_kernel


def _det_body(tag_scale, det_scale, preds_ref, heat_ref, mask_ref,
              part_ref, out_ref):
    b = pl.program_id(0)
    d = preds_ref[0]
    h = heat_ref[0]
    m = mask_ref[0]
    psum = jnp.sum((d - h[None]) ** 2 * m[None, None])

    @pl.when(b == 0)
    def _():
        out_ref[0, 0] = jnp.sum(part_ref[...]) * tag_scale

    out_ref[0, 0] = out_ref[0, 0] + psum * det_scale


def kernel(preds, masks, keypoints, gt_tags, heatmaps):
    loss_weights = (0.001, 1.0)
    B, nstack, chans, H, W = preds.shape
    n_parts = heatmaps.shape[1]
    tag_dim = gt_tags.shape[1]
    P, K = keypoints.shape[1], keypoints.shape[2]
    pk = P * K
    n_chunks = -(-pk // 128)
    pk_pad = n_chunks * 128
    n_workers = 32
    hw = H * W

    preds_flat = preds.reshape(-1)
    idx = keypoints[..., 0].astype(jnp.int32).reshape(B, pk)
    vis = keypoints[..., 1].astype(jnp.float32).reshape(B, pk)
    gt = gt_tags.astype(jnp.float32).reshape(B, pk)
    pad = ((0, 0), (0, pk_pad - pk))
    idx = jnp.pad(idx, pad).reshape(B, n_chunks, 128)
    vis = jnp.pad(vis, pad).reshape(B, n_chunks, 128)
    gt = jnp.pad(gt, pad).reshape(B, n_chunks, 128)

    tag_kernel = _make_tag_kernel(n_workers, n_chunks, chans, n_parts, hw)
    partials = tag_kernel(preds_flat, idx, gt, vis)

    tag_scale = loss_weights[0] / (B * nstack * tag_dim)
    det_scale = loss_weights[1] / (B * nstack * n_parts * H * W)
    out = pl.pallas_call(
        functools.partial(_det_body, tag_scale, det_scale),
        grid=(B,),
        in_specs=[
            pl.BlockSpec((1, nstack, n_parts, H, W), lambda b: (b, 0, 0, 0, 0)),
            pl.BlockSpec((1, n_parts, H, W), lambda b: (b, 0, 0, 0)),
            pl.BlockSpec((1, H, W), lambda b: (b, 0, 0)),
            pl.BlockSpec((n_workers, _LANES), lambda b: (0, 0)),
        ],
        out_specs=pl.BlockSpec(memory_space=pltpu.SMEM),
        out_shape=jax.ShapeDtypeStruct((1, 1), jnp.float32),
        compiler_params=pltpu.CompilerParams(
            dimension_semantics=("arbitrary",)),
    )(preds, heatmaps, masks, partials)
    return out[0, 0]


# speedup vs baseline: 3.5576x; 1.0490x over previous
import functools

import jax
import jax.numpy as jnp
from jax import lax
from jax.experimental import pallas as pl
from jax.experimental.pallas import tpu as pltpu
from jax.experimental.pallas import tpu_sc as plsc

_LANES = 16


def _make_tag_kernel(n_workers, n_chunks, chans, n_parts, hw):
    mesh = plsc.VectorSubcoreMesh(core_axis_name="c", subcore_axis_name="s")

    @functools.partial(
        pl.kernel,
        mesh=mesh,
        out_type=jax.ShapeDtypeStruct((n_workers, _LANES), jnp.float32),
        scratch_types=[
            pltpu.VMEM((n_chunks, 128), jnp.int32),
            pltpu.VMEM((n_chunks, 128), jnp.int32),
            pltpu.VMEM((n_chunks, 128), jnp.float32),
            pltpu.VMEM((n_chunks, 128), jnp.float32),
            pltpu.VMEM((n_chunks, 128), jnp.float32),
            pltpu.VMEM((_LANES,), jnp.float32),
            pltpu.SemaphoreType.DMA,
        ],
    )
    def tag_kernel(preds_flat, kp_idx, gt, vis, out,
                   idx_v, addr_v, vals_v, gt_v, vis_v, acc_v, sem):
        wid = lax.axis_index("s") * 2 + lax.axis_index("c")
        b = wid // 2
        pltpu.sync_copy(kp_idx.at[b], idx_v)
        pltpu.sync_copy(gt.at[b], gt_v)
        pltpu.sync_copy(vis.at[b], vis_v)
        acc = jnp.zeros((_LANES,), jnp.float32)
        for t in range(2):
            j = wid * 2 + t
            base = (j * chans + n_parts) * hw
            for c in range(n_chunks):
                for i in range(128 // _LANES):
                    sl = pl.ds(i * _LANES, _LANES)
                    addr_v[c, sl] = idx_v[c, sl] + base
            copies = [
                pltpu.async_copy(preds_flat.at[addr_v.at[c]], vals_v.at[c], sem)
                for c in range(n_chunks)
            ]
            for cp in copies:
                cp.wait()
            for c in range(n_chunks):
                for i in range(128 // _LANES):
                    sl = pl.ds(i * _LANES, _LANES)
                    d = vals_v[c, sl] - gt_v[c, sl]
                    acc = acc + d * d * vis_v[c, sl]
        acc_v[...] = acc
        pltpu.sync_copy(acc_v, out.at[wid])

    return tag_kernel


def _det_body(tag_scale, det_scale, preds_ref, heat_ref, mask_ref,
              out_ref):
    b = pl.program_id(0)
    d = preds_ref[0]
    h = heat_ref[0]
    m = mask_ref[0]
    psum = jnp.sum((d - h[None]) ** 2 * m[None, None])

    @pl.when(b == 0)
    def _():
        out_ref[0, 0] = 0.0

    out_ref[0, 0] = out_ref[0, 0] + psum * det_scale


def kernel(preds, masks, keypoints, gt_tags, heatmaps):
    loss_weights = (0.001, 1.0)
    B, nstack, chans, H, W = preds.shape
    n_parts = heatmaps.shape[1]
    tag_dim = gt_tags.shape[1]
    P, K = keypoints.shape[1], keypoints.shape[2]
    pk = P * K
    n_chunks = -(-pk // 128)
    pk_pad = n_chunks * 128
    n_workers = 32
    hw = H * W

    preds_flat = preds.reshape(-1)
    idx = keypoints[..., 0].astype(jnp.int32).reshape(B, pk)
    vis = keypoints[..., 1].astype(jnp.float32).reshape(B, pk)
    gt = gt_tags.astype(jnp.float32).reshape(B, pk)
    pad = ((0, 0), (0, pk_pad - pk))
    idx = jnp.pad(idx, pad).reshape(B, n_chunks, 128)
    vis = jnp.pad(vis, pad).reshape(B, n_chunks, 128)
    gt = jnp.pad(gt, pad).reshape(B, n_chunks, 128)

    tag_kernel = _make_tag_kernel(n_workers, n_chunks, chans, n_parts, hw)
    partials = tag_kernel(preds_flat, idx, gt, vis)

    tag_scale = loss_weights[0] / (B * nstack * tag_dim)
    det_scale = loss_weights[1] / (B * nstack * n_parts * H * W)
    out = pl.pallas_call(
        functools.partial(_det_body, tag_scale, det_scale),
        grid=(B,),
        in_specs=[
            pl.BlockSpec((1, nstack, n_parts, H, W), lambda b: (b, 0, 0, 0, 0)),
            pl.BlockSpec((1, n_parts, H, W), lambda b: (b, 0, 0, 0)),
            pl.BlockSpec((1, H, W), lambda b: (b, 0, 0)),
        ],
        out_specs=pl.BlockSpec(memory_space=pltpu.SMEM),
        out_shape=jax.ShapeDtypeStruct((1, 1), jnp.float32),
        compiler_params=pltpu.CompilerParams(
            dimension_semantics=("arbitrary",)),
    )(preds, heatmaps, masks)
    return out[0, 0] + jnp.sum(partials) * tag_scale
